# baseline (speedup 1.0000x reference)
import jax
import jax.numpy as jnp
from jax.experimental import pallas as pl
from jax.experimental.pallas import tpu as pltpu

_S = 2
_N_PATCH = 64


def _fused_kernel(x0_ref, x1_ref, w_ref, b_ref, o_ref, *, L, T, C, E, K):
    M = 2 * T
    xr = jnp.concatenate([x0_ref[0], x1_ref[0]], axis=0)
    xt = xr.T.astype(jnp.bfloat16)
    rows = [xt]
    for k in range(1, K):
        rows.append(jnp.concatenate([xt[:, k:], xt[:, :k]], axis=1))
    Xc = jnp.concatenate(rows, axis=0)
    Y = jnp.dot(w_ref[...], Xc, preferred_element_type=jnp.float32)
    Y = Y + b_ref[...]
    z = Y[:E] * jax.nn.sigmoid(Y[E:])
    t_idx = jax.lax.broadcasted_iota(jnp.int32, (E, M), 1)
    keep = (t_idx % _S == 0) & ((t_idx % T) < _S * L)
    z = jnp.where(keep, z, -jnp.inf)
    zt = z.T
    o_ref[...] = zt.reshape(2, _N_PATCH, T // _N_PATCH, E).max(axis=2)


def kernel(x, W1, b1, W2, b2):
    B, T, C = x.shape
    E, _, K = W1.shape
    L = (T - K) // _S + 1

    def fold_w(W):
        return jnp.transpose(W, (0, 2, 1)).reshape(E, K * C)

    Wc = jnp.concatenate([fold_w(W1), fold_w(W2)], axis=0).astype(jnp.bfloat16)
    bc = jnp.concatenate([b1, b2]).reshape(2 * E, 1)

    out = pl.pallas_call(
        lambda x0, x1, wref, bref, oref: _fused_kernel(
            x0, x1, wref, bref, oref, L=L, T=T, C=C, E=E, K=K),
        grid=(B // 2,),
        in_specs=[
            pl.BlockSpec((1, T, C), lambda b: (2 * b, 0, 0)),
            pl.BlockSpec((1, T, C), lambda b: (2 * b + 1, 0, 0)),
            pl.BlockSpec((2 * E, K * C), lambda b: (0, 0)),
            pl.BlockSpec((2 * E, 1), lambda b: (0, 0)),
        ],
        out_specs=pl.BlockSpec((2, _N_PATCH, E), lambda b: (b, 0, 0)),
        out_shape=jax.ShapeDtypeStruct((B, _N_PATCH, E), jnp.float32),
        compiler_params=pltpu.CompilerParams(
            dimension_semantics=("parallel",)),
    )(x, x, Wc, bc)
    return out


# speedup vs baseline: 1.0070x; 1.0070x over previous
import jax
import jax.numpy as jnp
from jax.experimental import pallas as pl
from jax.experimental.pallas import tpu as pltpu

_S = 2
_N_PATCH = 64


def _fused_kernel(x_ref, w_ref, b_ref, o_ref, *, L, T, C, E, K, BPB):
    M = BPB * T
    xt = x_ref[...].reshape(M, C).T.astype(jnp.bfloat16)
    rows = [xt]
    for k in range(1, K):
        rows.append(jnp.concatenate([xt[:, k:], xt[:, :k]], axis=1))
    Xc = jnp.concatenate(rows, axis=0)
    Y = jnp.dot(w_ref[...], Xc, preferred_element_type=jnp.float32)
    Y = Y + b_ref[...]
    z = Y[:E] * jax.nn.sigmoid(Y[E:])
    t_idx = jax.lax.broadcasted_iota(jnp.int32, (E, M), 1)
    keep = (t_idx % _S == 0) & ((t_idx % T) < _S * L)
    z = jnp.where(keep, z, -jnp.inf)
    zt = z.T
    o_ref[...] = zt.reshape(BPB, _N_PATCH, T // _N_PATCH, E).max(axis=2)


def kernel(x, W1, b1, W2, b2):
    B, T, C = x.shape
    E, _, K = W1.shape
    L = (T - K) // _S + 1

    def fold_w(W):
        return jnp.transpose(W, (0, 2, 1)).reshape(E, K * C)

    Wc = jnp.concatenate([fold_w(W1), fold_w(W2)], axis=0).astype(jnp.bfloat16)
    bc = jnp.concatenate([b1, b2]).reshape(2 * E, 1)

    BPB = 2
    out = pl.pallas_call(
        lambda xref, wref, bref, oref: _fused_kernel(
            xref, wref, bref, oref, L=L, T=T, C=C, E=E, K=K, BPB=BPB),
        grid=(B // BPB,),
        in_specs=[
            pl.BlockSpec((BPB, T, C), lambda b: (b, 0, 0)),
            pl.BlockSpec((2 * E, K * C), lambda b: (0, 0)),
            pl.BlockSpec((2 * E, 1), lambda b: (0, 0)),
        ],
        out_specs=pl.BlockSpec((BPB, _N_PATCH, E), lambda b: (b, 0, 0)),
        out_shape=jax.ShapeDtypeStruct((B, _N_PATCH, E), jnp.float32),
        compiler_params=pltpu.CompilerParams(
            dimension_semantics=("parallel",)),
    )(x, Wc, bc)
    return out
